# baseline (speedup 1.0000x reference)
import jax
import jax.numpy as jnp
from jax import lax
from jax.experimental import pallas as pl
from jax.experimental.pallas import tpu as pltpu

B, R, N, D = 4, 4, 1024, 256
NTILE = 512
NT = N // NTILE
L = 2


def _finish_tile(scat_ref, wcat_ref, wl, bias, x_own, w0_ref, b0_ref, den):
    agg = jnp.dot(scat_ref[...], wcat_ref[wl, 0],
                  preferred_element_type=jnp.float32)
    h0 = lax.dot_general(x_own, w0_ref[wl, 0], (((1,), (1,)), ((), ())),
                         preferred_element_type=jnp.float32)
    return jnp.maximum((agg + bias + h0 + b0_ref[wl, 0]) / den, 0.0)


def _body(adj_ref, x_ref, xown_ref, wcat_ref, brm_ref, w0_ref, b0_ref,
          out0_ref, out1_ref,
          acache_ref, x1_ref, bias1_ref, den_ref,
          scat0_ref, scat1_ref, rsm_ref, dacc_ref):
    bb = pl.program_id(0)
    n = pl.program_id(1)
    r = pl.program_id(2)

    @pl.when(bb < B)
    def _layer0():
        bn = bb * NT + n
        idx = bn * R + r
        adj_blk = adj_ref[0, 0]
        rowsum = jnp.sum(adj_blk, axis=1, keepdims=True)
        adj_bf = adj_blk.astype(jnp.bfloat16)
        acache_ref[idx] = adj_bf

        @pl.when(r == 0)
        def _():
            rsm_ref[...] = jnp.zeros((NTILE, 128), jnp.float32)
            dacc_ref[...] = rowsum

        for k in range(R):
            @pl.when(r == k)
            def _():
                rsm_ref[:, k:k + 1] = rowsum

        @pl.when(r > 0)
        def _():
            dacc_ref[...] += rowsum

        s = jnp.dot(adj_bf, x_ref[0], preferred_element_type=jnp.float32)
        sbf = s.astype(jnp.bfloat16)
        for k in range(R):
            @pl.when(r == k)
            def _():
                scat0_ref[:, k * D:(k + 1) * D] = sbf

        @pl.when(r == R - 1)
        def _():
            den = dacc_ref[...] + 1.0
            den_ref[bn] = den
            rsm = rsm_ref[...]
            bias1_ref[bn] = jnp.dot(rsm, brm_ref[1, 0],
                                    preferred_element_type=jnp.float32)
            bias0 = jnp.dot(rsm, brm_ref[0, 0],
                            preferred_element_type=jnp.float32)
            out = _finish_tile(scat0_ref, wcat_ref, 0, bias0,
                               xown_ref[0], w0_ref, b0_ref, den)
            out0_ref[0] = out
            x1_ref[bb, pl.ds(n * NTILE, NTILE)] = out.astype(jnp.bfloat16)

    @pl.when(bb >= 1)
    def _layer1():
        bp = bb - 1
        bn = bp * NT + n
        idx = bn * R + r
        s = jnp.dot(acache_ref[idx], x1_ref[bp],
                    preferred_element_type=jnp.float32)
        sbf = s.astype(jnp.bfloat16)
        for k in range(R):
            @pl.when(r == k)
            def _():
                scat1_ref[:, k * D:(k + 1) * D] = sbf

        @pl.when(r == R - 1)
        def _():
            out = _finish_tile(scat1_ref, wcat_ref, 1, bias1_ref[bn],
                               x1_ref[bp, pl.ds(n * NTILE, NTILE)],
                               w0_ref, b0_ref, den_ref[bn])
            out1_ref[0] = out


@jax.jit
def kernel(nodes, adj, W0, b0, Wr, br):
    bf = jnp.bfloat16
    xbf = nodes.astype(bf)
    wcat = Wr.transpose(0, 1, 3, 2).reshape(L, 1, R * D, D).astype(bf)
    brm = jnp.zeros((L, 1, 128, D), jnp.float32).at[:, 0, :R, :].set(br)

    out0, out1 = pl.pallas_call(
        _body,
        grid=(B + 1, NT, R),
        in_specs=[
            pl.BlockSpec((1, 1, NTILE, N),
                         lambda bb, n, r: (jnp.minimum(bb, B - 1),
                                           jnp.where(bb < B, r, 0),
                                           jnp.where(bb < B, n, 0), 0)),
            pl.BlockSpec((1, N, D),
                         lambda bb, n, r: (jnp.minimum(bb, B - 1), 0, 0)),
            pl.BlockSpec((1, NTILE, D),
                         lambda bb, n, r: (jnp.minimum(bb, B - 1),
                                           jnp.where(bb < B, n, 0), 0)),
            pl.BlockSpec((L, 1, R * D, D), lambda bb, n, r: (0, 0, 0, 0)),
            pl.BlockSpec((L, 1, 128, D), lambda bb, n, r: (0, 0, 0, 0)),
            pl.BlockSpec((L, 1, D, D), lambda bb, n, r: (0, 0, 0, 0)),
            pl.BlockSpec((L, 1, 1, D), lambda bb, n, r: (0, 0, 0, 0)),
        ],
        out_specs=[
            pl.BlockSpec((1, NTILE, D),
                         lambda bb, n, r: (jnp.minimum(bb, B - 1),
                                           jnp.where(bb < B, n, NT - 1), 0)),
            pl.BlockSpec((1, NTILE, D),
                         lambda bb, n, r: (jnp.maximum(bb - 1, 0),
                                           jnp.where(bb >= 1, n, 0), 0)),
        ],
        out_shape=[
            jax.ShapeDtypeStruct((B, N, D), jnp.float32),
            jax.ShapeDtypeStruct((B, N, D), jnp.float32),
        ],
        scratch_shapes=[
            pltpu.VMEM((B * NT * R, NTILE, N), jnp.bfloat16),
            pltpu.VMEM((B, N, D), jnp.bfloat16),
            pltpu.VMEM((B * NT, NTILE, D), jnp.float32),
            pltpu.VMEM((B * NT, NTILE, 1), jnp.float32),
            pltpu.VMEM((NTILE, R * D), jnp.bfloat16),
            pltpu.VMEM((NTILE, R * D), jnp.bfloat16),
            pltpu.VMEM((NTILE, 128), jnp.float32),
            pltpu.VMEM((NTILE, 1), jnp.float32),
        ],
    )(adj, xbf, xbf, wcat, brm, W0[:, None].astype(bf),
      b0[:, None, None, :])
    return (out0, out1)


import functools

from jax.experimental.pallas import tpu_sc as plsc

_SC_ROWS = B * R * N
_NW = 32
_RPW = _SC_ROWS // _NW
_CH = 16
_NCH = _RPW // _CH


def _sc_rowsums(adj_flat):
    mesh = plsc.VectorSubcoreMesh(core_axis_name="c", subcore_axis_name="s")

    @functools.partial(
        pl.kernel, mesh=mesh,
        out_type=jax.ShapeDtypeStruct((_SC_ROWS * 16,), jnp.float32),
        scratch_types=[
            pltpu.VMEM((_CH * N,), jnp.float32),
            pltpu.VMEM((_RPW * 16,), jnp.float32),
        ],
    )
    def k(adj_hbm, out_hbm, buf, acc):
        wid = lax.axis_index("s") * 2 + lax.axis_index("c")
        base = wid * _RPW

        def chunk(i, carry):
            pltpu.sync_copy(adj_hbm.at[pl.ds((base + i * _CH) * N, _CH * N)],
                            buf)
            for j in range(_CH):
                rowacc = buf[pl.ds(j * N, 16)]
                for v in range(1, N // 16):
                    rowacc = rowacc + buf[pl.ds(j * N + v * 16, 16)]
                acc[pl.ds((i * _CH + j) * 16, 16)] = rowacc
            return carry

        lax.fori_loop(0, _NCH, chunk, 0)
        pltpu.sync_copy(acc, out_hbm.at[pl.ds(base * 16, _RPW * 16)])

    return k(adj_flat).reshape(_SC_ROWS, 16).sum(axis=1)


_kernel_tc = kernel


@jax.jit
def _kernel_probe(nodes, adj, W0, b0, Wr, br):
    out0, out1 = _kernel_tc(nodes, adj, W0, b0, Wr, br)
    rs = _sc_rowsums(adj.reshape(_SC_ROWS * N))
    gate = jnp.sum(rs) >= -1.0
    return (out0, jnp.where(gate, out1, 0.0))


kernel = _kernel_probe


# speedup vs baseline: 3.0016x; 3.0016x over previous
import jax
import jax.numpy as jnp
from jax import lax
from jax.experimental import pallas as pl
from jax.experimental.pallas import tpu as pltpu

B, R, N, D = 4, 4, 1024, 256
NTILE = 512
NT = N // NTILE
L = 2


def _finish_tile(scat_ref, wcat_ref, wl, bias, x_own, w0_ref, b0_ref, den):
    agg = jnp.dot(scat_ref[...], wcat_ref[wl, 0],
                  preferred_element_type=jnp.float32)
    h0 = lax.dot_general(x_own, w0_ref[wl, 0], (((1,), (1,)), ((), ())),
                         preferred_element_type=jnp.float32)
    return jnp.maximum((agg + bias + h0 + b0_ref[wl, 0]) / den, 0.0)


def _body(adj_ref, x_ref, xown_ref, wcat_ref, brm_ref, w0_ref, b0_ref,
          out0_ref, out1_ref,
          acache_ref, x1_ref, bias1_ref, den_ref,
          scat0_ref, scat1_ref, rsm_ref, dacc_ref):
    bb = pl.program_id(0)
    n = pl.program_id(1)
    r = pl.program_id(2)

    @pl.when(bb < B)
    def _layer0():
        bn = bb * NT + n
        idx = bn * R + r
        adj_blk = adj_ref[0, 0]
        rowsum = jnp.sum(adj_blk, axis=1, keepdims=True)
        adj_bf = adj_blk.astype(jnp.bfloat16)
        acache_ref[idx] = adj_bf

        @pl.when(r == 0)
        def _():
            rsm_ref[...] = jnp.zeros((NTILE, 128), jnp.float32)
            dacc_ref[...] = rowsum

        for k in range(R):
            @pl.when(r == k)
            def _():
                rsm_ref[:, k:k + 1] = rowsum

        @pl.when(r > 0)
        def _():
            dacc_ref[...] += rowsum

        s = jnp.dot(adj_bf, x_ref[0], preferred_element_type=jnp.float32)
        sbf = s.astype(jnp.bfloat16)
        for k in range(R):
            @pl.when(r == k)
            def _():
                scat0_ref[:, k * D:(k + 1) * D] = sbf

        @pl.when(r == R - 1)
        def _():
            den = dacc_ref[...] + 1.0
            den_ref[bn] = den
            rsm = rsm_ref[...]
            bias1_ref[bn] = jnp.dot(rsm, brm_ref[1, 0],
                                    preferred_element_type=jnp.float32)
            bias0 = jnp.dot(rsm, brm_ref[0, 0],
                            preferred_element_type=jnp.float32)
            out = _finish_tile(scat0_ref, wcat_ref, 0, bias0,
                               xown_ref[0], w0_ref, b0_ref, den)
            out0_ref[0] = out
            x1_ref[bb, pl.ds(n * NTILE, NTILE)] = out.astype(jnp.bfloat16)

    @pl.when(bb >= 1)
    def _layer1():
        bp = bb - 1
        bn = bp * NT + n
        idx = bn * R + r
        s = jnp.dot(acache_ref[idx], x1_ref[bp],
                    preferred_element_type=jnp.float32)
        sbf = s.astype(jnp.bfloat16)
        for k in range(R):
            @pl.when(r == k)
            def _():
                scat1_ref[:, k * D:(k + 1) * D] = sbf

        @pl.when(r == R - 1)
        def _():
            out = _finish_tile(scat1_ref, wcat_ref, 1, bias1_ref[bn],
                               x1_ref[bp, pl.ds(n * NTILE, NTILE)],
                               w0_ref, b0_ref, den_ref[bn])
            out1_ref[0] = out


@jax.jit
def kernel(nodes, adj, W0, b0, Wr, br):
    bf = jnp.bfloat16
    xbf = nodes.astype(bf)
    wcat = Wr.transpose(0, 1, 3, 2).reshape(L, 1, R * D, D).astype(bf)
    brm = jnp.zeros((L, 1, 128, D), jnp.float32).at[:, 0, :R, :].set(br)

    out0, out1 = pl.pallas_call(
        _body,
        grid=(B + 1, NT, R),
        in_specs=[
            pl.BlockSpec((1, 1, NTILE, N),
                         lambda bb, n, r: (jnp.minimum(bb, B - 1),
                                           jnp.where(bb < B, r, 0),
                                           jnp.where(bb < B, n, 0), 0)),
            pl.BlockSpec((1, N, D),
                         lambda bb, n, r: (jnp.minimum(bb, B - 1), 0, 0)),
            pl.BlockSpec((1, NTILE, D),
                         lambda bb, n, r: (jnp.minimum(bb, B - 1),
                                           jnp.where(bb < B, n, 0), 0)),
            pl.BlockSpec((L, 1, R * D, D), lambda bb, n, r: (0, 0, 0, 0)),
            pl.BlockSpec((L, 1, 128, D), lambda bb, n, r: (0, 0, 0, 0)),
            pl.BlockSpec((L, 1, D, D), lambda bb, n, r: (0, 0, 0, 0)),
            pl.BlockSpec((L, 1, 1, D), lambda bb, n, r: (0, 0, 0, 0)),
        ],
        out_specs=[
            pl.BlockSpec((1, NTILE, D),
                         lambda bb, n, r: (jnp.minimum(bb, B - 1),
                                           jnp.where(bb < B, n, NT - 1), 0)),
            pl.BlockSpec((1, NTILE, D),
                         lambda bb, n, r: (jnp.maximum(bb - 1, 0),
                                           jnp.where(bb >= 1, n, 0), 0)),
        ],
        out_shape=[
            jax.ShapeDtypeStruct((B, N, D), jnp.float32),
            jax.ShapeDtypeStruct((B, N, D), jnp.float32),
        ],
        scratch_shapes=[
            pltpu.VMEM((B * NT * R, NTILE, N), jnp.bfloat16),
            pltpu.VMEM((B, N, D), jnp.bfloat16),
            pltpu.VMEM((B * NT, NTILE, D), jnp.float32),
            pltpu.VMEM((B * NT, NTILE, 1), jnp.float32),
            pltpu.VMEM((NTILE, R * D), jnp.bfloat16),
            pltpu.VMEM((NTILE, R * D), jnp.bfloat16),
            pltpu.VMEM((NTILE, 128), jnp.float32),
            pltpu.VMEM((NTILE, 1), jnp.float32),
        ],
    )(adj, xbf, xbf, wcat, brm, W0[:, None].astype(bf),
      b0[:, None, None, :])
    return (out0, out1)
